# baseline (speedup 1.0000x reference)
import functools

import jax
import jax.numpy as jnp
from jax import lax
from jax.experimental import pallas as pl
from jax.experimental.pallas import tpu as pltpu
from jax.experimental.pallas import tpu_sc as plsc


def _gelu(x):
    c = 0.7978845608028654
    return 0.5 * x * (1.0 + jnp.tanh(c * (x + 0.044715 * x * x * x)))


def _linear_body(x_ref, w_ref, b_ref, o_ref, *, act):
    acc = jnp.dot(x_ref[...], w_ref[...], preferred_element_type=jnp.float32)
    acc = acc + b_ref[...]
    if act:
        acc = _gelu(acc)
    o_ref[...] = acc


def _fused_linear(x, W, b, act=False, bm=512, bn=1024):
    M, K = x.shape
    N = W.shape[1]
    bn = min(bn, N)
    Mp = ((M + bm - 1) // bm) * bm
    xp = jnp.pad(x, ((0, Mp - M), (0, 0))) if Mp != M else x
    b2 = b.reshape(1, N)
    out = pl.pallas_call(
        functools.partial(_linear_body, act=act),
        grid=(Mp // bm, N // bn),
        in_specs=[
            pl.BlockSpec((bm, K), lambda i, j: (i, 0)),
            pl.BlockSpec((K, bn), lambda i, j: (0, j)),
            pl.BlockSpec((1, bn), lambda i, j: (0, j)),
        ],
        out_specs=pl.BlockSpec((bm, bn), lambda i, j: (i, j)),
        out_shape=jax.ShapeDtypeStruct((Mp, N), jnp.float32),
    )(xp, W, b2)
    return out[:M] if Mp != M else out


def _linear_ln_body(h_ref, w_ref, b_ref, r_ref, g_ref, be_ref, o_ref):
    acc = jnp.dot(h_ref[...], w_ref[...], preferred_element_type=jnp.float32)
    acc = acc + b_ref[...] + r_ref[...]
    mu = acc.mean(-1, keepdims=True)
    var = ((acc - mu) ** 2).mean(-1, keepdims=True)
    o_ref[...] = (acc - mu) / jnp.sqrt(var + 1e-5) * g_ref[...] + be_ref[...]


def _fused_linear_res_ln(h, W, b, res, g, be, bm=512):
    M, K = h.shape
    N = W.shape[1]
    Mp = ((M + bm - 1) // bm) * bm
    if Mp != M:
        h = jnp.pad(h, ((0, Mp - M), (0, 0)))
        res = jnp.pad(res, ((0, Mp - M), (0, 0)))
    out = pl.pallas_call(
        _linear_ln_body,
        grid=(Mp // bm,),
        in_specs=[
            pl.BlockSpec((bm, K), lambda i: (i, 0)),
            pl.BlockSpec((K, N), lambda i: (0, 0)),
            pl.BlockSpec((1, N), lambda i: (0, 0)),
            pl.BlockSpec((bm, N), lambda i: (i, 0)),
            pl.BlockSpec((1, N), lambda i: (0, 0)),
            pl.BlockSpec((1, N), lambda i: (0, 0)),
        ],
        out_specs=pl.BlockSpec((bm, N), lambda i: (i, 0)),
        out_shape=jax.ShapeDtypeStruct((Mp, N), jnp.float32),
    )(h, W, b.reshape(1, N), res, g.reshape(1, N), be.reshape(1, N))
    return out[:M] if Mp != M else out


_GA = -1.5957691216057308
_GB = -0.07135481282553504

_HC = (0.00022580887889489532, 0.39700430631637573, -0.06403439491987228,
       0.008633735589683056, -0.0007942942902445793, 4.6354525693459436e-05,
       -1.5300794302675058e-06, 2.159141843094403e-08)


def _gelu_poly(pre):
    t = pre * pre
    h = jnp.float32(_HC[7])
    for cf in _HC[6::-1]:
        h = h * t + jnp.float32(cf)
    g = 0.5 * pre + h
    g = jnp.where(pre >= 4.0, pre, g)
    return jnp.where(pre <= -4.0, 0.0, g)


def _edge_msg_sum(A, Bm, Ce8, rowp, colp, etp, rowptr, N, D, BS=16):
    info = plsc.get_sparse_core_info()
    NC, NS = info.num_cores, info.num_subcores
    NW = NC * NS
    NCT = ((N + NW - 1) // NW + 7) // 8 * 8
    RP_LEN = NCT + 8
    NV = D // 16

    mesh = plsc.VectorSubcoreMesh(core_axis_name="c", subcore_axis_name="s")

    @functools.partial(
        pl.kernel, mesh=mesh,
        out_type=jax.ShapeDtypeStruct((N, D), jnp.float32),
        scratch_types=[
            pltpu.VMEM((RP_LEN + 16,), jnp.int32),
            pltpu.VMEM((BS + 16,), jnp.int32),
            pltpu.VMEM((BS + 16,), jnp.int32),
            pltpu.VMEM((BS + 16,), jnp.int32),
            pltpu.VMEM((BS, D), jnp.float32),
            pltpu.VMEM((BS, D), jnp.float32),
            pltpu.VMEM((8, D), jnp.float32),
            pltpu.VMEM((D,), jnp.float32),
            pltpu.SemaphoreType.DMA,
            pltpu.SemaphoreType.DMA,
        ],
    )
    def k(A_h, B_h, Ce_h, rowp_h, colp_h, etp_h, rp_h, S_h,
          rp_v, rowb_v, colb_v, etb_v, ar_v, br_v, ce_v, acc_v, sem_a, sem_b):
        wid = lax.axis_index("s") * NC + lax.axis_index("c")
        base_n = wid * NCT
        nn = jnp.minimum(NCT, N - base_n)
        pltpu.sync_copy(rp_h.at[pl.ds(base_n, RP_LEN)], rp_v.at[pl.ds(0, RP_LEN)])
        pltpu.sync_copy(Ce_h, ce_v)
        e_lo = rp_v[pl.ds(0, 16)][0]
        e_hi = rp_v[pl.ds(nn, 16)][0]
        e0 = (e_lo // BS) * BS
        nb = (e_hi - e0 + BS - 1) // BS
        zero = jnp.zeros((16,), jnp.float32)
        for f in range(NV):
            acc_v[pl.ds(f * 16, 16)] = zero

        def batch_body(bi, prev):
            eb = e0 + bi * BS
            pltpu.sync_copy(rowp_h.at[pl.ds(eb, BS)], rowb_v.at[pl.ds(0, BS)])
            pltpu.sync_copy(colp_h.at[pl.ds(eb, BS)], colb_v.at[pl.ds(0, BS)])
            pltpu.sync_copy(etp_h.at[pl.ds(eb, BS)], etb_v.at[pl.ds(0, BS)])
            ca = pltpu.async_copy(A_h.at[rowb_v.at[pl.ds(0, BS)]], ar_v, sem_a)
            cb = pltpu.async_copy(B_h.at[colb_v.at[pl.ds(0, BS)]], br_v, sem_b)
            ca.wait()
            cb.wait()

            def edge_body(k_, prev):
                e = eb + k_
                c = colb_v[pl.ds(k_, 16)][0]
                et = etb_v[pl.ds(k_, 16)][0]
                valid = jnp.logical_and(e >= e_lo, e < e_hi)
                flush = jnp.logical_and(valid,
                                        jnp.logical_and(prev >= 0, c != prev))

                @pl.when(flush)
                def _():
                    pltpu.sync_copy(acc_v, S_h.at[prev])
                    for f in range(NV):
                        acc_v[pl.ds(f * 16, 16)] = zero

                @pl.when(valid)
                def _():
                    for f in range(NV):
                        sl = pl.ds(f * 16, 16)
                        pre = ar_v[k_, sl] + br_v[k_, sl] + ce_v[et, sl]
                        acc_v[sl] = acc_v[sl] + _gelu_poly(pre)

                return jnp.where(valid, c, prev)

            return lax.fori_loop(0, BS, edge_body, prev)

        prev = lax.fori_loop(0, nb, batch_body, jnp.int32(-1))

        @pl.when(prev >= 0)
        def _():
            pltpu.sync_copy(acc_v, S_h.at[prev])

    return k(A, Bm, Ce8, rowp, colp, etp, rowptr)


def _scaled_linear_body(x_ref, s_ref, w_ref, o_ref):
    sc = s_ref[...]
    xs = jnp.where(sc > 0.0, x_ref[...] * sc, 0.0)
    o_ref[...] = jnp.dot(xs, w_ref[...], preferred_element_type=jnp.float32)


def _fused_scaled_linear(x, scale, W, bm=512):
    M, K = x.shape
    N = W.shape[1]
    Mp = ((M + bm - 1) // bm) * bm
    if Mp != M:
        x = jnp.pad(x, ((0, Mp - M), (0, 0)))
        scale = jnp.pad(scale, ((0, Mp - M), (0, 0)))
    out = pl.pallas_call(
        _scaled_linear_body,
        grid=(Mp // bm,),
        in_specs=[
            pl.BlockSpec((bm, K), lambda i: (i, 0)),
            pl.BlockSpec((bm, 1), lambda i: (i, 0)),
            pl.BlockSpec((K, N), lambda i: (0, 0)),
        ],
        out_specs=pl.BlockSpec((bm, N), lambda i: (i, 0)),
        out_shape=jax.ShapeDtypeStruct((Mp, N), jnp.float32),
    )(x, scale, W)
    return out[:M] if Mp != M else out


def kernel(atom_types, edge_index, edge_types, batch_idx, params):
    p = params
    x = _fused_linear(p['atom_emb'][atom_types], p['in_W'], p['in_b'])
    row, col = edge_index[0], edge_index[1]
    N = x.shape[0]
    E = row.shape[0]

    perm = jnp.argsort(col)
    rowp, colp, etp = row[perm], col[perm], edge_types[perm]

    cnt = jax.ops.segment_sum(jnp.ones((E,), jnp.float32), colp, num_segments=N,
                              indices_are_sorted=True)
    inv_cnt = (1.0 / jnp.clip(cnt, 1.0, None))[:, None]
    has_edge = (cnt > 0.0).astype(jnp.float32)[:, None]
    s_scale = jnp.where(has_edge > 0.0, inv_cnt, 0.0)

    info = plsc.get_sparse_core_info()
    NW = info.num_cores * info.num_subcores
    NCT = ((N + NW - 1) // NW + 7) // 8 * 8
    rowptr = jnp.concatenate([jnp.zeros((1,), jnp.int32),
                              jnp.cumsum(cnt.astype(jnp.int32))])
    rowptr = jnp.concatenate(
        [rowptr, jnp.full((NW * NCT + 8 - (N + 1),), E, jnp.int32)])
    BS = 16
    epad = jnp.zeros((BS,), jnp.int32)
    rowp_p = jnp.concatenate([rowp, epad])
    colp_p = jnp.concatenate([colp, epad])
    etp_p = jnp.concatenate([etp, epad])

    for lp in p['mpnn']:
        A = _fused_linear(x, lp['m1W'][:512], jnp.zeros((1024,), jnp.float32))
        Bm = _fused_linear(x, lp['m1W'][512:1024], jnp.zeros((1024,), jnp.float32))
        Ce = p['edge_emb'] @ lp['m1W'][1024:] + lp['m1b']
        Ce8 = jnp.pad(Ce, ((0, 3), (0, 0)))
        S = _edge_msg_sum(A, Bm, Ce8, rowp_p, colp_p, etp_p, rowptr, N, 1024, BS=BS)
        aggr = _fused_scaled_linear(S, s_scale, lp['m2W']) + has_edge * lp['m2b']
        hu = _fused_linear(jnp.concatenate([x, aggr], -1), lp['u1W'], lp['u1b'], act=True)
        x = _fused_linear_res_ln(hu, lp['u2W'], lp['u2b'], x, lp['g'], lp['be'])

    Bn = 2048
    pooled = jax.ops.segment_sum(x, batch_idx, num_segments=Bn)
    pcnt = jax.ops.segment_sum(jnp.ones((N,), jnp.float32), batch_idx, num_segments=Bn)
    mol = _fused_linear(pooled / jnp.clip(pcnt, 1.0, None)[:, None], p['out_W'], p['out_b'])
    h = _fused_linear(mol, p['proj_W'], p['proj_b'])
    for lp in p['hg']:
        f = _fused_linear(h, lp['f1W'], lp['f1b'], act=True)
        h = _fused_linear_res_ln(f, lp['f2W'], lp['f2b'], h, lp['g'], lp['be'])
    prod = _fused_linear(_fused_linear(h, p['pp1W'], p['pp1b'], act=True), p['pp2W'], p['pp2b'])
    co = _fused_linear(_fused_linear(h, p['cp1W'], p['cp1b'], act=True), p['cp2W'], p['cp2b'])
    return (prod, co)


# speedup vs baseline: 3.5904x; 3.5904x over previous
import functools

import jax
import jax.numpy as jnp
from jax import lax
from jax.experimental import pallas as pl
from jax.experimental.pallas import tpu as pltpu
from jax.experimental.pallas import tpu_sc as plsc


def _gelu(x):
    c = 0.7978845608028654
    return 0.5 * x * (1.0 + jnp.tanh(c * (x + 0.044715 * x * x * x)))


def _linear_body(x_ref, w_ref, b_ref, o_ref, *, act):
    acc = jnp.dot(x_ref[...], w_ref[...], preferred_element_type=jnp.float32)
    acc = acc + b_ref[...]
    if act:
        acc = _gelu(acc)
    o_ref[...] = acc


def _fused_linear(x, W, b, act=False, bm=512, bn=1024):
    M, K = x.shape
    N = W.shape[1]
    bn = min(bn, N)
    Mp = ((M + bm - 1) // bm) * bm
    xp = jnp.pad(x, ((0, Mp - M), (0, 0))) if Mp != M else x
    b2 = b.reshape(1, N)
    out = pl.pallas_call(
        functools.partial(_linear_body, act=act),
        grid=(Mp // bm, N // bn),
        in_specs=[
            pl.BlockSpec((bm, K), lambda i, j: (i, 0)),
            pl.BlockSpec((K, bn), lambda i, j: (0, j)),
            pl.BlockSpec((1, bn), lambda i, j: (0, j)),
        ],
        out_specs=pl.BlockSpec((bm, bn), lambda i, j: (i, j)),
        out_shape=jax.ShapeDtypeStruct((Mp, N), jnp.float32),
    )(xp, W, b2)
    return out[:M] if Mp != M else out


def _linear_ln_body(h_ref, w_ref, b_ref, r_ref, g_ref, be_ref, o_ref):
    acc = jnp.dot(h_ref[...], w_ref[...], preferred_element_type=jnp.float32)
    acc = acc + b_ref[...] + r_ref[...]
    mu = acc.mean(-1, keepdims=True)
    var = ((acc - mu) ** 2).mean(-1, keepdims=True)
    o_ref[...] = (acc - mu) / jnp.sqrt(var + 1e-5) * g_ref[...] + be_ref[...]


def _fused_linear_res_ln(h, W, b, res, g, be, bm=512):
    M, K = h.shape
    N = W.shape[1]
    Mp = ((M + bm - 1) // bm) * bm
    if Mp != M:
        h = jnp.pad(h, ((0, Mp - M), (0, 0)))
        res = jnp.pad(res, ((0, Mp - M), (0, 0)))
    out = pl.pallas_call(
        _linear_ln_body,
        grid=(Mp // bm,),
        in_specs=[
            pl.BlockSpec((bm, K), lambda i: (i, 0)),
            pl.BlockSpec((K, N), lambda i: (0, 0)),
            pl.BlockSpec((1, N), lambda i: (0, 0)),
            pl.BlockSpec((bm, N), lambda i: (i, 0)),
            pl.BlockSpec((1, N), lambda i: (0, 0)),
            pl.BlockSpec((1, N), lambda i: (0, 0)),
        ],
        out_specs=pl.BlockSpec((bm, N), lambda i: (i, 0)),
        out_shape=jax.ShapeDtypeStruct((Mp, N), jnp.float32),
    )(h, W, b.reshape(1, N), res, g.reshape(1, N), be.reshape(1, N))
    return out[:M] if Mp != M else out


_GA = -1.5957691216057308
_GB = -0.07135481282553504

_HC = (0.00022580887889489532, 0.39700430631637573, -0.06403439491987228,
       0.008633735589683056, -0.0007942942902445793, 4.6354525693459436e-05,
       -1.5300794302675058e-06, 2.159141843094403e-08)


def _gelu_poly(pre):
    t = pre * pre
    h = jnp.float32(_HC[7])
    for cf in _HC[6::-1]:
        h = h * t + jnp.float32(cf)
    g = 0.5 * pre + h
    g = jnp.where(pre >= 4.0, pre, g)
    return jnp.where(pre <= -4.0, 0.0, g)


def _edge_msg_sum(A, Bm, Ce8, rowp, colp, etp, rowptr, N, D, BS=16):
    info = plsc.get_sparse_core_info()
    NC, NS = info.num_cores, info.num_subcores
    NW = NC * NS
    NCT = ((N + NW - 1) // NW + 7) // 8 * 8
    RP_LEN = NCT + 8
    NV = D // 16

    mesh = plsc.VectorSubcoreMesh(core_axis_name="c", subcore_axis_name="s")

    @functools.partial(
        pl.kernel, mesh=mesh,
        out_type=jax.ShapeDtypeStruct((N, D), jnp.float32),
        scratch_types=[
            pltpu.VMEM((RP_LEN + 16,), jnp.int32),
            pltpu.VMEM((BS + 16,), jnp.int32),
            pltpu.VMEM((BS + 16,), jnp.int32),
            pltpu.VMEM((BS + 16,), jnp.int32),
            pltpu.VMEM((BS, D), jnp.float32),
            pltpu.VMEM((BS, D), jnp.float32),
            pltpu.VMEM((8, D), jnp.float32),
            pltpu.VMEM((D,), jnp.float32),
            pltpu.SemaphoreType.DMA,
            pltpu.SemaphoreType.DMA,
        ],
    )
    def k(A_h, B_h, Ce_h, rowp_h, colp_h, etp_h, rp_h, S_h,
          rp_v, rowb_v, colb_v, etb_v, ar_v, br_v, ce_v, acc_v, sem_a, sem_b):
        wid = lax.axis_index("s") * NC + lax.axis_index("c")
        base_n = wid * NCT
        nn = jnp.minimum(NCT, N - base_n)
        pltpu.sync_copy(rp_h.at[pl.ds(base_n, RP_LEN)], rp_v.at[pl.ds(0, RP_LEN)])
        pltpu.sync_copy(Ce_h, ce_v)
        e_lo = rp_v[pl.ds(0, 16)][0]
        e_hi = rp_v[pl.ds(nn, 16)][0]
        e0 = (e_lo // BS) * BS
        nb = (e_hi - e0 + BS - 1) // BS
        zero = jnp.zeros((16,), jnp.float32)
        for f in range(NV):
            acc_v[pl.ds(f * 16, 16)] = zero

        def batch_body(bi, prev):
            eb = e0 + bi * BS
            pltpu.sync_copy(rowp_h.at[pl.ds(eb, BS)], rowb_v.at[pl.ds(0, BS)])
            pltpu.sync_copy(colp_h.at[pl.ds(eb, BS)], colb_v.at[pl.ds(0, BS)])
            pltpu.sync_copy(etp_h.at[pl.ds(eb, BS)], etb_v.at[pl.ds(0, BS)])
            ca = pltpu.async_copy(A_h.at[rowb_v.at[pl.ds(0, BS)]], ar_v, sem_a)
            cb = pltpu.async_copy(B_h.at[colb_v.at[pl.ds(0, BS)]], br_v, sem_b)
            ca.wait()
            cb.wait()

            def edge_body(k_, prev):
                e = eb + k_
                c = colb_v[pl.ds(k_, 16)][0]
                et = etb_v[pl.ds(k_, 16)][0]
                valid = jnp.logical_and(e >= e_lo, e < e_hi)
                flush = jnp.logical_and(valid,
                                        jnp.logical_and(prev >= 0, c != prev))

                @pl.when(flush)
                def _():
                    pltpu.sync_copy(acc_v, S_h.at[prev])
                    for f in range(NV):
                        acc_v[pl.ds(f * 16, 16)] = zero

                @pl.when(valid)
                def _():
                    U = 8
                    for f0 in range(0, NV, U):
                        sls = [pl.ds((f0 + j) * 16, 16) for j in range(U)]
                        pres = [ar_v[k_, s] + br_v[k_, s] + ce_v[et, s]
                                for s in sls]
                        ms = [p * (_GA + _GB * (p * p)) for p in pres]
                        es = [jnp.exp(m) for m in ms]
                        gs = [p / (1.0 + e) for p, e in zip(pres, es)]
                        for s, g in zip(sls, gs):
                            acc_v[s] = acc_v[s] + g

                return jnp.where(valid, c, prev)

            return lax.fori_loop(0, BS, edge_body, prev)

        prev = lax.fori_loop(0, nb, batch_body, jnp.int32(-1))

        @pl.when(prev >= 0)
        def _():
            pltpu.sync_copy(acc_v, S_h.at[prev])

    return k(A, Bm, Ce8, rowp, colp, etp, rowptr)


def _scaled_linear_body(x_ref, s_ref, w_ref, o_ref):
    sc = s_ref[...]
    xs = jnp.where(sc > 0.0, x_ref[...] * sc, 0.0)
    o_ref[...] = jnp.dot(xs, w_ref[...], preferred_element_type=jnp.float32)


def _fused_scaled_linear(x, scale, W, bm=512):
    M, K = x.shape
    N = W.shape[1]
    Mp = ((M + bm - 1) // bm) * bm
    if Mp != M:
        x = jnp.pad(x, ((0, Mp - M), (0, 0)))
        scale = jnp.pad(scale, ((0, Mp - M), (0, 0)))
    out = pl.pallas_call(
        _scaled_linear_body,
        grid=(Mp // bm,),
        in_specs=[
            pl.BlockSpec((bm, K), lambda i: (i, 0)),
            pl.BlockSpec((bm, 1), lambda i: (i, 0)),
            pl.BlockSpec((K, N), lambda i: (0, 0)),
        ],
        out_specs=pl.BlockSpec((bm, N), lambda i: (i, 0)),
        out_shape=jax.ShapeDtypeStruct((Mp, N), jnp.float32),
    )(x, scale, W)
    return out[:M] if Mp != M else out


def kernel(atom_types, edge_index, edge_types, batch_idx, params):
    p = params
    x = _fused_linear(p['atom_emb'][atom_types], p['in_W'], p['in_b'])
    row, col = edge_index[0], edge_index[1]
    N = x.shape[0]
    E = row.shape[0]

    perm = jnp.argsort(col)
    rowp, colp, etp = row[perm], col[perm], edge_types[perm]

    cnt = jax.ops.segment_sum(jnp.ones((E,), jnp.float32), colp, num_segments=N,
                              indices_are_sorted=True)
    inv_cnt = (1.0 / jnp.clip(cnt, 1.0, None))[:, None]
    has_edge = (cnt > 0.0).astype(jnp.float32)[:, None]
    s_scale = jnp.where(has_edge > 0.0, inv_cnt, 0.0)

    info = plsc.get_sparse_core_info()
    NW = info.num_cores * info.num_subcores
    NCT = ((N + NW - 1) // NW + 7) // 8 * 8
    rowptr = jnp.concatenate([jnp.zeros((1,), jnp.int32),
                              jnp.cumsum(cnt.astype(jnp.int32))])
    rowptr = jnp.concatenate(
        [rowptr, jnp.full((NW * NCT + 8 - (N + 1),), E, jnp.int32)])
    BS = 16
    epad = jnp.zeros((BS,), jnp.int32)
    rowp_p = jnp.concatenate([rowp, epad])
    colp_p = jnp.concatenate([colp, epad])
    etp_p = jnp.concatenate([etp, epad])

    for lp in p['mpnn']:
        A = _fused_linear(x, lp['m1W'][:512], jnp.zeros((1024,), jnp.float32))
        Bm = _fused_linear(x, lp['m1W'][512:1024], jnp.zeros((1024,), jnp.float32))
        Ce = p['edge_emb'] @ lp['m1W'][1024:] + lp['m1b']
        Ce8 = jnp.pad(Ce, ((0, 3), (0, 0)))
        S = _edge_msg_sum(A, Bm, Ce8, rowp_p, colp_p, etp_p, rowptr, N, 1024, BS=BS)
        aggr = _fused_scaled_linear(S, s_scale, lp['m2W']) + has_edge * lp['m2b']
        hu = _fused_linear(jnp.concatenate([x, aggr], -1), lp['u1W'], lp['u1b'], act=True)
        x = _fused_linear_res_ln(hu, lp['u2W'], lp['u2b'], x, lp['g'], lp['be'])

    Bn = 2048
    pooled = jax.ops.segment_sum(x, batch_idx, num_segments=Bn)
    pcnt = jax.ops.segment_sum(jnp.ones((N,), jnp.float32), batch_idx, num_segments=Bn)
    mol = _fused_linear(pooled / jnp.clip(pcnt, 1.0, None)[:, None], p['out_W'], p['out_b'])
    h = _fused_linear(mol, p['proj_W'], p['proj_b'])
    for lp in p['hg']:
        f = _fused_linear(h, lp['f1W'], lp['f1b'], act=True)
        h = _fused_linear_res_ln(f, lp['f2W'], lp['f2b'], h, lp['g'], lp['be'])
    prod = _fused_linear(_fused_linear(h, p['pp1W'], p['pp1b'], act=True), p['pp2W'], p['pp2b'])
    co = _fused_linear(_fused_linear(h, p['cp1W'], p['cp1b'], act=True), p['cp2W'], p['cp2b'])
    return (prod, co)


# speedup vs baseline: 3.5921x; 1.0005x over previous
import functools

import jax
import jax.numpy as jnp
from jax import lax
from jax.experimental import pallas as pl
from jax.experimental.pallas import tpu as pltpu
from jax.experimental.pallas import tpu_sc as plsc


def _gelu(x):
    c = 0.7978845608028654
    return 0.5 * x * (1.0 + jnp.tanh(c * (x + 0.044715 * x * x * x)))


def _linear_body(x_ref, w_ref, b_ref, o_ref, *, act, lowp):
    xv, wv = x_ref[...], w_ref[...]
    if lowp:
        xv, wv = xv.astype(jnp.bfloat16), wv.astype(jnp.bfloat16)
    acc = jnp.dot(xv, wv, preferred_element_type=jnp.float32)
    acc = acc + b_ref[...]
    if act:
        acc = _gelu(acc)
    o_ref[...] = acc


def _fused_linear(x, W, b, act=False, bm=512, bn=1024, lowp=False):
    M, K = x.shape
    N = W.shape[1]
    bn = min(bn, N)
    Mp = ((M + bm - 1) // bm) * bm
    xp = jnp.pad(x, ((0, Mp - M), (0, 0))) if Mp != M else x
    b2 = b.reshape(1, N)
    out = pl.pallas_call(
        functools.partial(_linear_body, act=act, lowp=lowp),
        grid=(Mp // bm, N // bn),
        in_specs=[
            pl.BlockSpec((bm, K), lambda i, j: (i, 0)),
            pl.BlockSpec((K, bn), lambda i, j: (0, j)),
            pl.BlockSpec((1, bn), lambda i, j: (0, j)),
        ],
        out_specs=pl.BlockSpec((bm, bn), lambda i, j: (i, j)),
        out_shape=jax.ShapeDtypeStruct((Mp, N), jnp.float32),
    )(xp, W, b2)
    return out[:M] if Mp != M else out


def _linear_ln_body(h_ref, w_ref, b_ref, r_ref, g_ref, be_ref, o_ref):
    acc = jnp.dot(h_ref[...], w_ref[...], preferred_element_type=jnp.float32)
    acc = acc + b_ref[...] + r_ref[...]
    mu = acc.mean(-1, keepdims=True)
    var = ((acc - mu) ** 2).mean(-1, keepdims=True)
    o_ref[...] = (acc - mu) / jnp.sqrt(var + 1e-5) * g_ref[...] + be_ref[...]


def _fused_linear_res_ln(h, W, b, res, g, be, bm=512):
    M, K = h.shape
    N = W.shape[1]
    Mp = ((M + bm - 1) // bm) * bm
    if Mp != M:
        h = jnp.pad(h, ((0, Mp - M), (0, 0)))
        res = jnp.pad(res, ((0, Mp - M), (0, 0)))
    out = pl.pallas_call(
        _linear_ln_body,
        grid=(Mp // bm,),
        in_specs=[
            pl.BlockSpec((bm, K), lambda i: (i, 0)),
            pl.BlockSpec((K, N), lambda i: (0, 0)),
            pl.BlockSpec((1, N), lambda i: (0, 0)),
            pl.BlockSpec((bm, N), lambda i: (i, 0)),
            pl.BlockSpec((1, N), lambda i: (0, 0)),
            pl.BlockSpec((1, N), lambda i: (0, 0)),
        ],
        out_specs=pl.BlockSpec((bm, N), lambda i: (i, 0)),
        out_shape=jax.ShapeDtypeStruct((Mp, N), jnp.float32),
    )(h, W, b.reshape(1, N), res, g.reshape(1, N), be.reshape(1, N))
    return out[:M] if Mp != M else out


_GA = -1.5957691216057308
_GB = -0.07135481282553504

_HC = (0.00022580887889489532, 0.39700430631637573, -0.06403439491987228,
       0.008633735589683056, -0.0007942942902445793, 4.6354525693459436e-05,
       -1.5300794302675058e-06, 2.159141843094403e-08)


def _gelu_poly(pre):
    t = pre * pre
    h = jnp.float32(_HC[7])
    for cf in _HC[6::-1]:
        h = h * t + jnp.float32(cf)
    g = 0.5 * pre + h
    g = jnp.where(pre >= 4.0, pre, g)
    return jnp.where(pre <= -4.0, 0.0, g)


def _edge_msg_sum(A, Bm, Ce8, rowp, colp, etp, rowptr, N, D, BS=16):
    info = plsc.get_sparse_core_info()
    NC, NS = info.num_cores, info.num_subcores
    NW = NC * NS
    NCT = ((N + NW - 1) // NW + 7) // 8 * 8
    RP_LEN = NCT + 8
    NV = D // 16

    mesh = plsc.VectorSubcoreMesh(core_axis_name="c", subcore_axis_name="s")

    @functools.partial(
        pl.kernel, mesh=mesh,
        out_type=jax.ShapeDtypeStruct((N, D), jnp.float32),
        scratch_types=[
            pltpu.VMEM((RP_LEN + 16,), jnp.int32),
            pltpu.VMEM((BS + 16,), jnp.int32),
            pltpu.VMEM((BS + 16,), jnp.int32),
            pltpu.VMEM((BS + 16,), jnp.int32),
            pltpu.VMEM((BS, D), jnp.float32),
            pltpu.VMEM((BS, D), jnp.float32),
            pltpu.VMEM((8, D), jnp.float32),
            pltpu.VMEM((D,), jnp.float32),
            pltpu.SemaphoreType.DMA,
            pltpu.SemaphoreType.DMA,
        ],
    )
    def k(A_h, B_h, Ce_h, rowp_h, colp_h, etp_h, rp_h, S_h,
          rp_v, rowb_v, colb_v, etb_v, ar_v, br_v, ce_v, acc_v, sem_a, sem_b):
        wid = lax.axis_index("s") * NC + lax.axis_index("c")
        base_n = wid * NCT
        nn = jnp.minimum(NCT, N - base_n)
        pltpu.sync_copy(rp_h.at[pl.ds(base_n, RP_LEN)], rp_v.at[pl.ds(0, RP_LEN)])
        pltpu.sync_copy(Ce_h, ce_v)
        e_lo = rp_v[pl.ds(0, 16)][0]
        e_hi = rp_v[pl.ds(nn, 16)][0]
        e0 = (e_lo // BS) * BS
        nb = (e_hi - e0 + BS - 1) // BS
        zero = jnp.zeros((16,), jnp.float32)
        for f in range(NV):
            acc_v[pl.ds(f * 16, 16)] = zero

        def batch_body(bi, prev):
            eb = e0 + bi * BS
            pltpu.sync_copy(rowp_h.at[pl.ds(eb, BS)], rowb_v.at[pl.ds(0, BS)])
            pltpu.sync_copy(colp_h.at[pl.ds(eb, BS)], colb_v.at[pl.ds(0, BS)])
            pltpu.sync_copy(etp_h.at[pl.ds(eb, BS)], etb_v.at[pl.ds(0, BS)])
            ca = pltpu.async_copy(A_h.at[rowb_v.at[pl.ds(0, BS)]], ar_v, sem_a)
            cb = pltpu.async_copy(B_h.at[colb_v.at[pl.ds(0, BS)]], br_v, sem_b)
            ca.wait()
            cb.wait()

            def edge_body(k_, prev):
                e = eb + k_
                c = colb_v[pl.ds(k_, 16)][0]
                et = etb_v[pl.ds(k_, 16)][0]
                valid = jnp.logical_and(e >= e_lo, e < e_hi)
                flush = jnp.logical_and(valid,
                                        jnp.logical_and(prev >= 0, c != prev))

                @pl.when(flush)
                def _():
                    pltpu.sync_copy(acc_v, S_h.at[prev])
                    for f in range(NV):
                        acc_v[pl.ds(f * 16, 16)] = zero

                @pl.when(valid)
                def _():
                    U = 8
                    for f0 in range(0, NV, U):
                        sls = [pl.ds((f0 + j) * 16, 16) for j in range(U)]
                        pres = [ar_v[k_, s] + br_v[k_, s] + ce_v[et, s]
                                for s in sls]
                        ms = [p * (_GA + _GB * (p * p)) for p in pres]
                        es = [jnp.exp(m) for m in ms]
                        gs = [p / (1.0 + e) for p, e in zip(pres, es)]
                        for s, g in zip(sls, gs):
                            acc_v[s] = acc_v[s] + g

                return jnp.where(valid, c, prev)

            return lax.fori_loop(0, BS, edge_body, prev)

        prev = lax.fori_loop(0, nb, batch_body, jnp.int32(-1))

        @pl.when(prev >= 0)
        def _():
            pltpu.sync_copy(acc_v, S_h.at[prev])

    return k(A, Bm, Ce8, rowp, colp, etp, rowptr)


def _scaled_linear_body(x_ref, s_ref, w_ref, o_ref):
    sc = s_ref[...]
    xs = jnp.where(sc > 0.0, x_ref[...] * sc, 0.0)
    o_ref[...] = jnp.dot(xs.astype(jnp.bfloat16),
                         w_ref[...].astype(jnp.bfloat16),
                         preferred_element_type=jnp.float32)


def _fused_scaled_linear(x, scale, W, bm=512):
    M, K = x.shape
    N = W.shape[1]
    Mp = ((M + bm - 1) // bm) * bm
    if Mp != M:
        x = jnp.pad(x, ((0, Mp - M), (0, 0)))
        scale = jnp.pad(scale, ((0, Mp - M), (0, 0)))
    out = pl.pallas_call(
        _scaled_linear_body,
        grid=(Mp // bm,),
        in_specs=[
            pl.BlockSpec((bm, K), lambda i: (i, 0)),
            pl.BlockSpec((bm, 1), lambda i: (i, 0)),
            pl.BlockSpec((K, N), lambda i: (0, 0)),
        ],
        out_specs=pl.BlockSpec((bm, N), lambda i: (i, 0)),
        out_shape=jax.ShapeDtypeStruct((Mp, N), jnp.float32),
    )(x, scale, W)
    return out[:M] if Mp != M else out


def kernel(atom_types, edge_index, edge_types, batch_idx, params):
    p = params
    x = _fused_linear(p['atom_emb'][atom_types], p['in_W'], p['in_b'])
    row, col = edge_index[0], edge_index[1]
    N = x.shape[0]
    E = row.shape[0]

    perm = jnp.argsort(col)
    rowp, colp, etp = row[perm], col[perm], edge_types[perm]

    cnt = jax.ops.segment_sum(jnp.ones((E,), jnp.float32), colp, num_segments=N,
                              indices_are_sorted=True)
    inv_cnt = (1.0 / jnp.clip(cnt, 1.0, None))[:, None]
    has_edge = (cnt > 0.0).astype(jnp.float32)[:, None]
    s_scale = jnp.where(has_edge > 0.0, inv_cnt, 0.0)

    info = plsc.get_sparse_core_info()
    NW = info.num_cores * info.num_subcores
    NCT = ((N + NW - 1) // NW + 7) // 8 * 8
    rowptr = jnp.concatenate([jnp.zeros((1,), jnp.int32),
                              jnp.cumsum(cnt.astype(jnp.int32))])
    rowptr = jnp.concatenate(
        [rowptr, jnp.full((NW * NCT + 8 - (N + 1),), E, jnp.int32)])
    BS = 16
    epad = jnp.zeros((BS,), jnp.int32)
    rowp_p = jnp.concatenate([rowp, epad])
    colp_p = jnp.concatenate([colp, epad])
    etp_p = jnp.concatenate([etp, epad])

    for lp in p['mpnn']:
        A = _fused_linear(x, lp['m1W'][:512], jnp.zeros((1024,), jnp.float32), lowp=True)
        Bm = _fused_linear(x, lp['m1W'][512:1024], jnp.zeros((1024,), jnp.float32), lowp=True)
        Ce = p['edge_emb'] @ lp['m1W'][1024:] + lp['m1b']
        Ce8 = jnp.pad(Ce, ((0, 3), (0, 0)))
        S = _edge_msg_sum(A, Bm, Ce8, rowp_p, colp_p, etp_p, rowptr, N, 1024, BS=BS)
        aggr = _fused_scaled_linear(S, s_scale, lp['m2W']) + has_edge * lp['m2b']
        hu = _fused_linear(jnp.concatenate([x, aggr], -1), lp['u1W'], lp['u1b'],
                           act=True, lowp=True)
        x = _fused_linear_res_ln(hu, lp['u2W'], lp['u2b'], x, lp['g'], lp['be'])

    Bn = 2048
    pooled = jax.ops.segment_sum(x, batch_idx, num_segments=Bn)
    pcnt = jax.ops.segment_sum(jnp.ones((N,), jnp.float32), batch_idx, num_segments=Bn)
    mol = _fused_linear(pooled / jnp.clip(pcnt, 1.0, None)[:, None], p['out_W'], p['out_b'])
    h = _fused_linear(mol, p['proj_W'], p['proj_b'])
    for lp in p['hg']:
        f = _fused_linear(h, lp['f1W'], lp['f1b'], act=True)
        h = _fused_linear_res_ln(f, lp['f2W'], lp['f2b'], h, lp['g'], lp['be'])
    prod = _fused_linear(_fused_linear(h, p['pp1W'], p['pp1b'], act=True), p['pp2W'], p['pp2b'])
    co = _fused_linear(_fused_linear(h, p['cp1W'], p['cp1b'], act=True), p['cp2W'], p['cp2b'])
    return (prod, co)


# speedup vs baseline: 3.5962x; 1.0011x over previous
import functools

import jax
import jax.numpy as jnp
from jax import lax
from jax.experimental import pallas as pl
from jax.experimental.pallas import tpu as pltpu
from jax.experimental.pallas import tpu_sc as plsc


def _gelu(x):
    c = 0.7978845608028654
    return 0.5 * x * (1.0 + jnp.tanh(c * (x + 0.044715 * x * x * x)))


def _linear_body(x_ref, w_ref, b_ref, o_ref, *, act, lowp):
    xv, wv = x_ref[...], w_ref[...]
    if lowp:
        xv, wv = xv.astype(jnp.bfloat16), wv.astype(jnp.bfloat16)
    acc = jnp.dot(xv, wv, preferred_element_type=jnp.float32)
    acc = acc + b_ref[...]
    if act:
        acc = _gelu(acc)
    o_ref[...] = acc


def _fused_linear(x, W, b, act=False, bm=512, bn=1024, lowp=False):
    M, K = x.shape
    N = W.shape[1]
    bn = min(bn, N)
    Mp = ((M + bm - 1) // bm) * bm
    xp = jnp.pad(x, ((0, Mp - M), (0, 0))) if Mp != M else x
    b2 = b.reshape(1, N)
    out = pl.pallas_call(
        functools.partial(_linear_body, act=act, lowp=lowp),
        grid=(Mp // bm, N // bn),
        in_specs=[
            pl.BlockSpec((bm, K), lambda i, j: (i, 0)),
            pl.BlockSpec((K, bn), lambda i, j: (0, j)),
            pl.BlockSpec((1, bn), lambda i, j: (0, j)),
        ],
        out_specs=pl.BlockSpec((bm, bn), lambda i, j: (i, j)),
        out_shape=jax.ShapeDtypeStruct((Mp, N), jnp.float32),
    )(xp, W, b2)
    return out[:M] if Mp != M else out


def _linear_ln_body(h_ref, w_ref, b_ref, r_ref, g_ref, be_ref, o_ref):
    acc = jnp.dot(h_ref[...], w_ref[...], preferred_element_type=jnp.float32)
    acc = acc + b_ref[...] + r_ref[...]
    mu = acc.mean(-1, keepdims=True)
    var = ((acc - mu) ** 2).mean(-1, keepdims=True)
    o_ref[...] = (acc - mu) / jnp.sqrt(var + 1e-5) * g_ref[...] + be_ref[...]


def _fused_linear_res_ln(h, W, b, res, g, be, bm=512):
    M, K = h.shape
    N = W.shape[1]
    Mp = ((M + bm - 1) // bm) * bm
    if Mp != M:
        h = jnp.pad(h, ((0, Mp - M), (0, 0)))
        res = jnp.pad(res, ((0, Mp - M), (0, 0)))
    out = pl.pallas_call(
        _linear_ln_body,
        grid=(Mp // bm,),
        in_specs=[
            pl.BlockSpec((bm, K), lambda i: (i, 0)),
            pl.BlockSpec((K, N), lambda i: (0, 0)),
            pl.BlockSpec((1, N), lambda i: (0, 0)),
            pl.BlockSpec((bm, N), lambda i: (i, 0)),
            pl.BlockSpec((1, N), lambda i: (0, 0)),
            pl.BlockSpec((1, N), lambda i: (0, 0)),
        ],
        out_specs=pl.BlockSpec((bm, N), lambda i: (i, 0)),
        out_shape=jax.ShapeDtypeStruct((Mp, N), jnp.float32),
    )(h, W, b.reshape(1, N), res, g.reshape(1, N), be.reshape(1, N))
    return out[:M] if Mp != M else out


_GA = -1.5957691216057308
_GB = -0.07135481282553504

_HC = (0.00022580887889489532, 0.39700430631637573, -0.06403439491987228,
       0.008633735589683056, -0.0007942942902445793, 4.6354525693459436e-05,
       -1.5300794302675058e-06, 2.159141843094403e-08)


def _gelu_poly(pre):
    t = pre * pre
    h = jnp.float32(_HC[7])
    for cf in _HC[6::-1]:
        h = h * t + jnp.float32(cf)
    g = 0.5 * pre + h
    g = jnp.where(pre >= 4.0, pre, g)
    return jnp.where(pre <= -4.0, 0.0, g)


def _edge_msg_sum(A, Bm, Ce8, rowp, colp, etp, rowptr, N, D, BS=16):
    info = plsc.get_sparse_core_info()
    NC, NS = info.num_cores, info.num_subcores
    NW = NC * NS
    CH = 48
    NCT = ((N + NW - 1) // NW + CH - 1) // CH * CH
    NCHUNK = NCT // CH
    RP_LEN = NCT + 8
    NV = D // 16
    Np = NW * NCT

    mesh = plsc.VectorSubcoreMesh(core_axis_name="c", subcore_axis_name="s")

    @functools.partial(
        pl.kernel, mesh=mesh,
        out_type=jax.ShapeDtypeStruct((Np, D), jnp.float32),
        scratch_types=[
            pltpu.VMEM((RP_LEN + 16,), jnp.int32),
            pltpu.VMEM((BS + 16,), jnp.int32),
            pltpu.VMEM((BS + 16,), jnp.int32),
            pltpu.VMEM((BS + 16,), jnp.int32),
            pltpu.VMEM((BS, D), jnp.float32),
            pltpu.VMEM((BS, D), jnp.float32),
            pltpu.VMEM((8, D), jnp.float32),
            pltpu.VMEM((CH, D), jnp.float32),
            pltpu.SemaphoreType.DMA,
            pltpu.SemaphoreType.DMA,
        ],
    )
    def k(A_h, B_h, Ce_h, rowp_h, colp_h, etp_h, rp_h, S_h,
          rp_v, rowb_v, colb_v, etb_v, ar_v, br_v, ce_v, acc_v, sem_a, sem_b):
        wid = lax.axis_index("s") * NC + lax.axis_index("c")
        base_n = wid * NCT
        pltpu.sync_copy(rp_h.at[pl.ds(base_n, RP_LEN)], rp_v.at[pl.ds(0, RP_LEN)])
        pltpu.sync_copy(Ce_h, ce_v)
        zero = jnp.zeros((16,), jnp.float32)

        def chunk_body(ch, _):
            cbl = ch * CH
            cbg = base_n + cbl
            e_lo = rp_v[pl.ds(cbl, 16)][0]
            e_hi = rp_v[pl.ds(cbl + CH, 16)][0]

            def row_zero(r, _):
                for f in range(NV):
                    acc_v[r, pl.ds(f * 16, 16)] = zero
                return 0
            lax.fori_loop(0, CH, row_zero, 0)

            e0 = (e_lo // BS) * BS
            nb = (e_hi - e0 + BS - 1) // BS

            def batch_body(bi, _):
                eb = e0 + bi * BS
                pltpu.sync_copy(rowp_h.at[pl.ds(eb, BS)], rowb_v.at[pl.ds(0, BS)])
                pltpu.sync_copy(colp_h.at[pl.ds(eb, BS)], colb_v.at[pl.ds(0, BS)])
                pltpu.sync_copy(etp_h.at[pl.ds(eb, BS)], etb_v.at[pl.ds(0, BS)])
                ca = pltpu.async_copy(A_h.at[rowb_v.at[pl.ds(0, BS)]], ar_v, sem_a)
                cb = pltpu.async_copy(B_h.at[colb_v.at[pl.ds(0, BS)]], br_v, sem_b)
                ca.wait()
                cb.wait()

                def edge_body(k_, _):
                    e = eb + k_
                    c = colb_v[pl.ds(k_, 16)][0]
                    et = etb_v[pl.ds(k_, 16)][0]
                    ci = c - cbg
                    valid = jnp.logical_and(e >= e_lo, e < e_hi)

                    @pl.when(valid)
                    def _():
                        U = 8
                        for f0 in range(0, NV, U):
                            sls = [pl.ds((f0 + j) * 16, 16) for j in range(U)]
                            pres = [ar_v[k_, s] + br_v[k_, s] + ce_v[et, s]
                                    for s in sls]
                            ms = [p * (_GA + _GB * (p * p)) for p in pres]
                            es = [jnp.exp(m) for m in ms]
                            gs = [p / (1.0 + e_) for p, e_ in zip(pres, es)]
                            for s, g in zip(sls, gs):
                                acc_v[ci, s] = acc_v[ci, s] + g
                    return 0

                return lax.fori_loop(0, BS, edge_body, 0)

            lax.fori_loop(0, nb, batch_body, 0)
            pltpu.sync_copy(acc_v, S_h.at[pl.ds(cbg, CH)])
            return 0

        lax.fori_loop(0, NCHUNK, chunk_body, 0)

    return k(A, Bm, Ce8, rowp, colp, etp, rowptr)


def _scaled_linear_body(x_ref, s_ref, w_ref, o_ref):
    sc = s_ref[...]
    xs = jnp.where(sc > 0.0, x_ref[...] * sc, 0.0)
    o_ref[...] = jnp.dot(xs.astype(jnp.bfloat16),
                         w_ref[...].astype(jnp.bfloat16),
                         preferred_element_type=jnp.float32)


def _fused_scaled_linear(x, scale, W, bm=512):
    M, K = x.shape
    N = W.shape[1]
    Mp = ((M + bm - 1) // bm) * bm
    if Mp != M:
        x = jnp.pad(x, ((0, Mp - M), (0, 0)))
        scale = jnp.pad(scale, ((0, Mp - M), (0, 0)))
    out = pl.pallas_call(
        _scaled_linear_body,
        grid=(Mp // bm,),
        in_specs=[
            pl.BlockSpec((bm, K), lambda i: (i, 0)),
            pl.BlockSpec((bm, 1), lambda i: (i, 0)),
            pl.BlockSpec((K, N), lambda i: (0, 0)),
        ],
        out_specs=pl.BlockSpec((bm, N), lambda i: (i, 0)),
        out_shape=jax.ShapeDtypeStruct((Mp, N), jnp.float32),
    )(x, scale, W)
    return out[:M] if Mp != M else out


def kernel(atom_types, edge_index, edge_types, batch_idx, params):
    p = params
    x = _fused_linear(p['atom_emb'][atom_types], p['in_W'], p['in_b'])
    row, col = edge_index[0], edge_index[1]
    N = x.shape[0]
    E = row.shape[0]

    perm = jnp.argsort(col)
    rowp, colp, etp = row[perm], col[perm], edge_types[perm]

    cnt = jax.ops.segment_sum(jnp.ones((E,), jnp.float32), colp, num_segments=N,
                              indices_are_sorted=True)
    inv_cnt = (1.0 / jnp.clip(cnt, 1.0, None))[:, None]
    has_edge = (cnt > 0.0).astype(jnp.float32)[:, None]
    s_scale = jnp.where(has_edge > 0.0, inv_cnt, 0.0)

    info = plsc.get_sparse_core_info()
    NW = info.num_cores * info.num_subcores
    NCT = ((N + NW - 1) // NW + 47) // 48 * 48
    Np = NW * NCT
    rowptr = jnp.concatenate([jnp.zeros((1,), jnp.int32),
                              jnp.cumsum(cnt.astype(jnp.int32))])
    rowptr = jnp.concatenate(
        [rowptr, jnp.full((Np + 8 - (N + 1),), E, jnp.int32)])
    BS = 16
    epad = jnp.zeros((BS,), jnp.int32)
    rowp_p = jnp.concatenate([rowp, epad])
    colp_p = jnp.concatenate([colp, epad])
    etp_p = jnp.concatenate([etp, epad])

    for lp in p['mpnn']:
        A = _fused_linear(x, lp['m1W'][:512], jnp.zeros((1024,), jnp.float32), lowp=True)
        Bm = _fused_linear(x, lp['m1W'][512:1024], jnp.zeros((1024,), jnp.float32), lowp=True)
        Ce = p['edge_emb'] @ lp['m1W'][1024:] + lp['m1b']
        Ce8 = jnp.pad(Ce, ((0, 3), (0, 0)))
        S = _edge_msg_sum(A, Bm, Ce8, rowp_p, colp_p, etp_p, rowptr, N, 1024, BS=BS)
        sp = jnp.pad(s_scale, ((0, Np - N), (0, 0)))
        aggr = _fused_scaled_linear(S, sp, lp['m2W'])[:N] + has_edge * lp['m2b']
        hu = _fused_linear(jnp.concatenate([x, aggr], -1), lp['u1W'], lp['u1b'],
                           act=True, lowp=True)
        x = _fused_linear_res_ln(hu, lp['u2W'], lp['u2b'], x, lp['g'], lp['be'])

    Bn = 2048
    pooled = jax.ops.segment_sum(x, batch_idx, num_segments=Bn)
    pcnt = jax.ops.segment_sum(jnp.ones((N,), jnp.float32), batch_idx, num_segments=Bn)
    mol = _fused_linear(pooled / jnp.clip(pcnt, 1.0, None)[:, None], p['out_W'], p['out_b'])
    h = _fused_linear(mol, p['proj_W'], p['proj_b'])
    for lp in p['hg']:
        f = _fused_linear(h, lp['f1W'], lp['f1b'], act=True)
        h = _fused_linear_res_ln(f, lp['f2W'], lp['f2b'], h, lp['g'], lp['be'])
    prod = _fused_linear(_fused_linear(h, p['pp1W'], p['pp1b'], act=True), p['pp2W'], p['pp2b'])
    co = _fused_linear(_fused_linear(h, p['cp1W'], p['cp1b'], act=True), p['cp2W'], p['cp2b'])
    return (prod, co)


# speedup vs baseline: 3.7658x; 1.0472x over previous
import functools

import jax
import jax.numpy as jnp
from jax import lax
from jax.experimental import pallas as pl
from jax.experimental.pallas import tpu as pltpu
from jax.experimental.pallas import tpu_sc as plsc


def _gelu(x):
    c = 0.7978845608028654
    return 0.5 * x * (1.0 + jnp.tanh(c * (x + 0.044715 * x * x * x)))


def _linear_body(x_ref, w_ref, b_ref, o_ref, *, act, lowp):
    xv, wv = x_ref[...], w_ref[...]
    if lowp:
        xv, wv = xv.astype(jnp.bfloat16), wv.astype(jnp.bfloat16)
    acc = jnp.dot(xv, wv, preferred_element_type=jnp.float32)
    acc = acc + b_ref[...]
    if act:
        acc = _gelu(acc)
    o_ref[...] = acc


def _fused_linear(x, W, b, act=False, bm=512, bn=1024, lowp=False):
    M, K = x.shape
    N = W.shape[1]
    bn = min(bn, N)
    Mp = ((M + bm - 1) // bm) * bm
    xp = jnp.pad(x, ((0, Mp - M), (0, 0))) if Mp != M else x
    b2 = b.reshape(1, N)
    out = pl.pallas_call(
        functools.partial(_linear_body, act=act, lowp=lowp),
        grid=(Mp // bm, N // bn),
        in_specs=[
            pl.BlockSpec((bm, K), lambda i, j: (i, 0)),
            pl.BlockSpec((K, bn), lambda i, j: (0, j)),
            pl.BlockSpec((1, bn), lambda i, j: (0, j)),
        ],
        out_specs=pl.BlockSpec((bm, bn), lambda i, j: (i, j)),
        out_shape=jax.ShapeDtypeStruct((Mp, N), jnp.float32),
    )(xp, W, b2)
    return out[:M] if Mp != M else out


def _linear_ln_body(h_ref, w_ref, b_ref, r_ref, g_ref, be_ref, o_ref):
    acc = jnp.dot(h_ref[...], w_ref[...], preferred_element_type=jnp.float32)
    acc = acc + b_ref[...] + r_ref[...]
    mu = acc.mean(-1, keepdims=True)
    var = ((acc - mu) ** 2).mean(-1, keepdims=True)
    o_ref[...] = (acc - mu) / jnp.sqrt(var + 1e-5) * g_ref[...] + be_ref[...]


def _fused_linear_res_ln(h, W, b, res, g, be, bm=512):
    M, K = h.shape
    N = W.shape[1]
    Mp = ((M + bm - 1) // bm) * bm
    if Mp != M:
        h = jnp.pad(h, ((0, Mp - M), (0, 0)))
        res = jnp.pad(res, ((0, Mp - M), (0, 0)))
    out = pl.pallas_call(
        _linear_ln_body,
        grid=(Mp // bm,),
        in_specs=[
            pl.BlockSpec((bm, K), lambda i: (i, 0)),
            pl.BlockSpec((K, N), lambda i: (0, 0)),
            pl.BlockSpec((1, N), lambda i: (0, 0)),
            pl.BlockSpec((bm, N), lambda i: (i, 0)),
            pl.BlockSpec((1, N), lambda i: (0, 0)),
            pl.BlockSpec((1, N), lambda i: (0, 0)),
        ],
        out_specs=pl.BlockSpec((bm, N), lambda i: (i, 0)),
        out_shape=jax.ShapeDtypeStruct((Mp, N), jnp.float32),
    )(h, W, b.reshape(1, N), res, g.reshape(1, N), be.reshape(1, N))
    return out[:M] if Mp != M else out


_GA = -1.5957691216057308
_GB = -0.07135481282553504

_HC = (0.00022580887889489532, 0.39700430631637573, -0.06403439491987228,
       0.008633735589683056, -0.0007942942902445793, 4.6354525693459436e-05,
       -1.5300794302675058e-06, 2.159141843094403e-08)


def _gelu_poly(pre):
    t = pre * pre
    h = jnp.float32(_HC[7])
    for cf in _HC[6::-1]:
        h = h * t + jnp.float32(cf)
    g = 0.5 * pre + h
    g = jnp.where(pre >= 4.0, pre, g)
    return jnp.where(pre <= -4.0, 0.0, g)


def _edge_msg_sum(A, Bm, Ce8, rowp, colp, etp, rowptr, N, D, BS=16):
    info = plsc.get_sparse_core_info()
    NC, NS = info.num_cores, info.num_subcores
    NW = NC * NS
    CH = 48
    NCT = ((N + NW - 1) // NW + CH - 1) // CH * CH
    NCHUNK = NCT // CH
    RP_LEN = NCT + 8
    NV = D // 16
    Np = NW * NCT

    mesh = plsc.VectorSubcoreMesh(core_axis_name="c", subcore_axis_name="s")

    @functools.partial(
        pl.kernel, mesh=mesh,
        out_type=jax.ShapeDtypeStruct((Np, D), jnp.float32),
        scratch_types=[
            pltpu.VMEM((RP_LEN + 16,), jnp.int32),
            pltpu.VMEM((BS + 16,), jnp.int32),
            pltpu.VMEM((BS + 16,), jnp.int32),
            pltpu.VMEM((BS + 16,), jnp.int32),
            pltpu.VMEM((BS, D), jnp.float32),
            pltpu.VMEM((BS, D), jnp.float32),
            pltpu.VMEM((8, D), jnp.float32),
            pltpu.VMEM((CH, D), jnp.float32),
            pltpu.SemaphoreType.DMA,
            pltpu.SemaphoreType.DMA,
        ],
    )
    def k(A_h, B_h, Ce_h, rowp_h, colp_h, etp_h, rp_h, S_h,
          rp_v, rowb_v, colb_v, etb_v, ar_v, br_v, ce_v, acc_v, sem_a, sem_b):
        wid = lax.axis_index("s") * NC + lax.axis_index("c")
        base_n = wid * NCT
        pltpu.sync_copy(rp_h.at[pl.ds(base_n, RP_LEN)], rp_v.at[pl.ds(0, RP_LEN)])
        pltpu.sync_copy(Ce_h, ce_v)
        zero = jnp.zeros((16,), jnp.float32)

        def chunk_body(ch, _):
            cbl = ch * CH
            cbg = base_n + cbl
            e_lo = rp_v[pl.ds(cbl, 16)][0]
            e_hi = rp_v[pl.ds(cbl + CH, 16)][0]

            def row_zero(r, _):
                for f in range(NV):
                    acc_v[r, pl.ds(f * 16, 16)] = zero
                return 0
            lax.fori_loop(0, CH, row_zero, 0)

            e0 = (e_lo // BS) * BS
            nb = (e_hi - e0 + BS - 1) // BS

            def batch_body(bi, _):
                eb = e0 + bi * BS
                pltpu.sync_copy(rowp_h.at[pl.ds(eb, BS)], rowb_v.at[pl.ds(0, BS)])
                pltpu.sync_copy(colp_h.at[pl.ds(eb, BS)], colb_v.at[pl.ds(0, BS)])
                pltpu.sync_copy(etp_h.at[pl.ds(eb, BS)], etb_v.at[pl.ds(0, BS)])
                ca = pltpu.async_copy(A_h.at[rowb_v.at[pl.ds(0, BS)]], ar_v, sem_a)
                cb = pltpu.async_copy(B_h.at[colb_v.at[pl.ds(0, BS)]], br_v, sem_b)
                ca.wait()
                cb.wait()

                def edge_body(k_, _):
                    e = eb + k_
                    c = colb_v[pl.ds(k_, 16)][0]
                    et = etb_v[pl.ds(k_, 16)][0]
                    ci = c - cbg
                    valid = jnp.logical_and(e >= e_lo, e < e_hi)

                    @pl.when(valid)
                    def _():
                        U = 8
                        for f0 in range(0, NV, U):
                            sls = [pl.ds((f0 + j) * 16, 16) for j in range(U)]
                            pres = [ar_v[k_, s] + br_v[k_, s] + ce_v[et, s]
                                    for s in sls]
                            ms = [p * (_GA + _GB * (p * p)) for p in pres]
                            es = [jnp.exp(m) for m in ms]
                            gs = [p / (1.0 + e_) for p, e_ in zip(pres, es)]
                            for s, g in zip(sls, gs):
                                acc_v[ci, s] = acc_v[ci, s] + g
                    return 0

                return lax.fori_loop(0, BS, edge_body, 0)

            lax.fori_loop(0, nb, batch_body, 0)
            pltpu.sync_copy(acc_v, S_h.at[pl.ds(cbg, CH)])
            return 0

        lax.fori_loop(0, NCHUNK, chunk_body, 0)

    return k(A, Bm, Ce8, rowp, colp, etp, rowptr)


def _scaled_linear_body(x_ref, s_ref, w_ref, o_ref):
    sc = s_ref[...]
    xs = jnp.where(sc > 0.0, x_ref[...] * sc, 0.0)
    o_ref[...] = jnp.dot(xs.astype(jnp.bfloat16),
                         w_ref[...].astype(jnp.bfloat16),
                         preferred_element_type=jnp.float32)


def _fused_scaled_linear(x, scale, W, bm=512):
    M, K = x.shape
    N = W.shape[1]
    Mp = ((M + bm - 1) // bm) * bm
    if Mp != M:
        x = jnp.pad(x, ((0, Mp - M), (0, 0)))
        scale = jnp.pad(scale, ((0, Mp - M), (0, 0)))
    out = pl.pallas_call(
        _scaled_linear_body,
        grid=(Mp // bm,),
        in_specs=[
            pl.BlockSpec((bm, K), lambda i: (i, 0)),
            pl.BlockSpec((bm, 1), lambda i: (i, 0)),
            pl.BlockSpec((K, N), lambda i: (0, 0)),
        ],
        out_specs=pl.BlockSpec((bm, N), lambda i: (i, 0)),
        out_shape=jax.ShapeDtypeStruct((Mp, N), jnp.float32),
    )(x, scale, W)
    return out[:M] if Mp != M else out


def kernel(atom_types, edge_index, edge_types, batch_idx, params):
    p = params
    x = _fused_linear(p['atom_emb'][atom_types], p['in_W'], p['in_b'])
    row, col = edge_index[0], edge_index[1]
    N = x.shape[0]
    E = row.shape[0]

    perm = jnp.argsort(col)
    rowp, colp, etp = row[perm], col[perm], edge_types[perm]

    cnt = jax.ops.segment_sum(jnp.ones((E,), jnp.float32), colp, num_segments=N,
                              indices_are_sorted=True)
    inv_cnt = (1.0 / jnp.clip(cnt, 1.0, None))[:, None]
    has_edge = (cnt > 0.0).astype(jnp.float32)[:, None]
    s_scale = jnp.where(has_edge > 0.0, inv_cnt, 0.0)

    info = plsc.get_sparse_core_info()
    NW = info.num_cores * info.num_subcores
    NCT = ((N + NW - 1) // NW + 47) // 48 * 48
    Np = NW * NCT
    rowptr = jnp.concatenate([jnp.zeros((1,), jnp.int32),
                              jnp.cumsum(cnt.astype(jnp.int32))])
    rowptr = jnp.concatenate(
        [rowptr, jnp.full((Np + 8 - (N + 1),), E, jnp.int32)])
    BS = 32
    epad = jnp.zeros((BS,), jnp.int32)
    rowp_p = jnp.concatenate([rowp, epad])
    colp_p = jnp.concatenate([colp, epad])
    etp_p = jnp.concatenate([etp, epad])

    for lp in p['mpnn']:
        A = _fused_linear(x, lp['m1W'][:512], jnp.zeros((1024,), jnp.float32), lowp=True)
        Bm = _fused_linear(x, lp['m1W'][512:1024], jnp.zeros((1024,), jnp.float32), lowp=True)
        Ce = p['edge_emb'] @ lp['m1W'][1024:] + lp['m1b']
        Ce8 = jnp.pad(Ce, ((0, 3), (0, 0)))
        S = _edge_msg_sum(A, Bm, Ce8, rowp_p, colp_p, etp_p, rowptr, N, 1024, BS=BS)
        sp = jnp.pad(s_scale, ((0, Np - N), (0, 0)))
        aggr = _fused_scaled_linear(S, sp, lp['m2W'])[:N] + has_edge * lp['m2b']
        hu = _fused_linear(jnp.concatenate([x, aggr], -1), lp['u1W'], lp['u1b'],
                           act=True, lowp=True)
        x = _fused_linear_res_ln(hu, lp['u2W'], lp['u2b'], x, lp['g'], lp['be'])

    Bn = 2048
    pooled = jax.ops.segment_sum(x, batch_idx, num_segments=Bn)
    pcnt = jax.ops.segment_sum(jnp.ones((N,), jnp.float32), batch_idx, num_segments=Bn)
    mol = _fused_linear(pooled / jnp.clip(pcnt, 1.0, None)[:, None], p['out_W'], p['out_b'])
    h = _fused_linear(mol, p['proj_W'], p['proj_b'])
    for lp in p['hg']:
        f = _fused_linear(h, lp['f1W'], lp['f1b'], act=True)
        h = _fused_linear_res_ln(f, lp['f2W'], lp['f2b'], h, lp['g'], lp['be'])
    prod = _fused_linear(_fused_linear(h, p['pp1W'], p['pp1b'], act=True), p['pp2W'], p['pp2b'])
    co = _fused_linear(_fused_linear(h, p['cp1W'], p['cp1b'], act=True), p['cp2W'], p['cp2b'])
    return (prod, co)
